# baseline (speedup 1.0000x reference)
import functools

import jax
import jax.numpy as jnp
from jax import lax
from jax.experimental import pallas as pl
from jax.experimental.pallas import tpu as pltpu
from jax.experimental.pallas import tpu_sc as plsc

R, C = 4096, 8192
NC, NS, L = 2, 16, 16
NW = NC * NS
ROWS_PER_W = R // NW
ROWS_SUB = 4
NCHUNK = ROWS_PER_W // ROWS_SUB
VREGS = C // L

_MESH = plsc.VectorSubcoreMesh(core_axis_name="c", subcore_axis_name="s")


@functools.partial(
    pl.kernel,
    out_type=jax.ShapeDtypeStruct((R, C), jnp.float32),
    mesh=_MESH,
    scratch_types=[pltpu.MemorySpace.VMEM((ROWS_SUB, C), jnp.float32)],
    compiler_params=pltpu.CompilerParams(
        use_tc_tiling_on_sc=False, needs_layout_passes=False
    ),
)
def _cumsum_sc(x_hbm, out_hbm, buf):
    wid = lax.axis_index("s") * NC + lax.axis_index("c")

    def do_chunk(k, _):
        r0 = wid * ROWS_PER_W + k * ROWS_SUB
        pltpu.sync_copy(x_hbm.at[pl.ds(r0, ROWS_SUB), :], buf)

        def do_vreg(j, carries):
            c0 = j * L
            new = []
            for r in range(ROWS_SUB):
                v = buf[r, pl.ds(c0, L)]
                s = plsc.cumsum(v)
                t = jnp.sum(v)
                buf[r, pl.ds(c0, L)] = s + carries[r]
                new.append(carries[r] + t)
            return tuple(new)

        lax.fori_loop(0, VREGS, do_vreg,
                      (jnp.float32(0.0),) * ROWS_SUB, unroll=2)
        pltpu.sync_copy(buf, out_hbm.at[pl.ds(r0, ROWS_SUB), :])
        return 0

    lax.fori_loop(0, NCHUNK, do_chunk, 0)


def kernel(x):
    return _cumsum_sc(x)
